# baseline (speedup 1.0000x reference)
import functools

import jax
import jax.numpy as jnp
from jax import lax
from jax.experimental import pallas as pl
from jax.experimental.pallas import tpu as pltpu
from jax.experimental.pallas import tpu_sc as plsc

MAX_DEGREE = 512
HIDDEN_DIM = 128
N_NODES = 100000
N_EDGES = 1600000

NC = 2
NS = 16
NW = NC * NS
L = 16

EROWS = N_EDGES // 128
HIST_PAD = 100352
HSLICE = HIST_PAD // NS
NCHUNK = 782
TAIL_CHUNK = NCHUNK - 1
TAIL_BASE = TAIL_CHUNK * 128
TAIL_N = N_NODES - TAIL_BASE

_mesh = functools.partial(
    plsc.VectorSubcoreMesh,
    core_axis_name="c", subcore_axis_name="s",
    num_cores=NC, num_subcores=NS,
)


def _hist_body(edges_hbm, hist_hbm, zbuf, idx_v, ones_v, sh_src, sh_tgt):
    cid = lax.axis_index("c")
    sid = lax.axis_index("s")
    wid = sid * NC + cid

    @pl.loop(0, HSLICE // L)
    def _(i):
        zbuf[pl.ds(i * L, L)] = jnp.zeros((L,), jnp.int32)

    for j in range(128 // L):
        ones_v[pl.ds(j * L, L)] = jnp.ones((L,), jnp.int32)

    pltpu.sync_copy(zbuf, sh_src.at[pl.ds(sid * HSLICE, HSLICE)])
    pltpu.sync_copy(zbuf, sh_tgt.at[pl.ds(sid * HSLICE, HSLICE)])
    plsc.subcore_barrier()

    nfull = EROWS // NW
    nrows = jnp.where(wid < EROWS - nfull * NW, nfull + 1, nfull)

    @pl.loop(0, nrows)
    def _(i):
        r = wid + i * NW
        pltpu.sync_copy(edges_hbm.at[0, r], idx_v)
        pltpu.sync_copy(ones_v, sh_src.at[idx_v], add=True)
        pltpu.sync_copy(edges_hbm.at[1, r], idx_v)
        pltpu.sync_copy(ones_v, sh_tgt.at[idx_v], add=True)

    plsc.subcore_barrier()

    sl = pl.ds(sid * HSLICE, HSLICE)
    pltpu.sync_copy(sh_src.at[sl], hist_hbm.at[cid, 0, sl])
    pltpu.sync_copy(sh_tgt.at[sl], hist_hbm.at[cid, 1, sl])


_hist_call = pl.kernel(
    _hist_body,
    out_type=jax.ShapeDtypeStruct((NC, 2, HIST_PAD), jnp.int32),
    mesh=_mesh(),
    scratch_types=[
        pltpu.VMEM((HSLICE,), jnp.int32),
        pltpu.VMEM((128,), jnp.int32),
        pltpu.VMEM((128,), jnp.int32),
        pltpu.VMEM_SHARED((HIST_PAD,), jnp.int32),
        pltpu.VMEM_SHARED((HIST_PAD,), jnp.int32),
    ],
)


def _lookup_body(hist_hbm, in_emb, out_emb, off_hbm, out_hbm,
                 t0, t1, idx_in, idx_out, rows_a, rows_b, offv, sem_a, sem_b):
    cid = lax.axis_index("c")
    sid = lax.axis_index("s")
    wid = sid * NC + cid

    pltpu.sync_copy(off_hbm, offv)
    off_vec = offv[...]

    nfull = NCHUNK // NW
    nchunks = jnp.where(wid < NCHUNK - nfull * NW, nfull + 1, nfull)

    @pl.loop(0, nchunks)
    def _(i):
        g = wid + i * NW
        base = g * 128

        pltpu.sync_copy(hist_hbm.at[0, 1, pl.ds(base, 128)], t0)
        pltpu.sync_copy(hist_hbm.at[1, 1, pl.ds(base, 128)], t1)
        for j in range(128 // L):
            sl = pl.ds(j * L, L)
            v = t0[sl] + t1[sl] + off_vec
            idx_in[sl] = jnp.minimum(jnp.maximum(v, 0), MAX_DEGREE)
        pltpu.sync_copy(hist_hbm.at[0, 0, pl.ds(base, 128)], t0)
        pltpu.sync_copy(hist_hbm.at[1, 0, pl.ds(base, 128)], t1)
        for j in range(128 // L):
            sl = pl.ds(j * L, L)
            v = t0[sl] + t1[sl] + off_vec
            idx_out[sl] = jnp.minimum(jnp.maximum(v, 0), MAX_DEGREE)

        cp_a = pltpu.async_copy(in_emb.at[idx_in], rows_a, sem_a)
        cp_b = pltpu.async_copy(out_emb.at[idx_out], rows_b, sem_b)
        cp_a.wait()
        cp_b.wait()

        @pl.loop(0, 128)
        def _(r):
            for j in range(HIDDEN_DIM // L):
                sl = pl.ds(j * L, L)
                rows_a[r, sl] = rows_a[r, sl] + rows_b[r, sl]

        @pl.when(g < TAIL_CHUNK)
        def _():
            pltpu.sync_copy(rows_a, out_hbm.at[pl.ds(base, 128)])

        @pl.when(g == TAIL_CHUNK)
        def _():
            pltpu.sync_copy(rows_a.at[pl.ds(0, TAIL_N)],
                            out_hbm.at[pl.ds(TAIL_BASE, TAIL_N)])


_lookup_call = pl.kernel(
    _lookup_body,
    out_type=jax.ShapeDtypeStruct((N_NODES, HIDDEN_DIM), jnp.float32),
    mesh=_mesh(),
    scratch_types=[
        pltpu.VMEM((128,), jnp.int32),
        pltpu.VMEM((128,), jnp.int32),
        pltpu.VMEM((128,), jnp.int32),
        pltpu.VMEM((128,), jnp.int32),
        pltpu.VMEM((128, HIDDEN_DIM), jnp.float32),
        pltpu.VMEM((128, HIDDEN_DIM), jnp.float32),
        pltpu.VMEM((L,), jnp.int32),
        pltpu.SemaphoreType.DMA,
        pltpu.SemaphoreType.DMA,
    ],
)


def kernel(edge_index, n_nodes, in_embed, out_embed):
    edges = edge_index.reshape(2, EROWS, 128)
    off = (jnp.asarray(n_nodes) - N_NODES).astype(jnp.int32)
    off_v = jnp.full((L,), off, jnp.int32)
    hist = _hist_call(edges)
    return _lookup_call(hist, in_embed, out_embed, off_v)


# speedup vs baseline: 1.5806x; 1.5806x over previous
import functools

import jax
import jax.numpy as jnp
from jax import lax
from jax.experimental import pallas as pl
from jax.experimental.pallas import tpu as pltpu
from jax.experimental.pallas import tpu_sc as plsc

MAX_DEGREE = 512
HIDDEN_DIM = 128
N_NODES = 100000
N_EDGES = 1600000

NC = 2
NS = 16
NW = NC * NS
L = 16

EROWS = N_EDGES // 128
BR = 32
NBLK = EROWS // BR
REM_ROWS = EROWS - NBLK * BR
NBLK_MAX = -(-NBLK // NW) + 1
HIST_PAD = 100352
HSLICE = HIST_PAD // NS
NCHUNK = 782
NCHUNK_MAX = -(-NCHUNK // NW)
TAIL_CHUNK = NCHUNK - 1
TAIL_WID = TAIL_CHUNK % NW
TAIL_BASE = TAIL_CHUNK * 128
TAIL_N = N_NODES - TAIL_BASE

_mesh = functools.partial(
    plsc.VectorSubcoreMesh,
    core_axis_name="c", subcore_axis_name="s",
    num_cores=NC, num_subcores=NS,
)


def _hist_body(edges_hbm, hist_hbm, zbuf, eb0, eb1, ones_v, rbuf,
               sh_src, sh_tgt, dsem, ssem):
    cid = lax.axis_index("c")
    sid = lax.axis_index("s")
    wid = sid * NC + cid

    @pl.loop(0, HSLICE // L)
    def _(i):
        zbuf[pl.ds(i * L, L)] = jnp.zeros((L,), jnp.int32)

    for j in range(128 // L):
        ones_v[pl.ds(j * L, L)] = jnp.ones((L,), jnp.int32)

    pltpu.sync_copy(zbuf, sh_src.at[pl.ds(sid * HSLICE, HSLICE)])
    pltpu.sync_copy(zbuf, sh_tgt.at[pl.ds(sid * HSLICE, HSLICE)])
    plsc.subcore_barrier()

    nfull = NBLK // NW
    n = jnp.where(wid < NBLK - nfull * NW, nfull + 1, nfull)
    ebufs = (eb0, eb1)

    for kind, sh in ((0, sh_src), (1, sh_tgt)):
        @pl.when(n > 0)
        def _():
            row0 = pl.multiple_of(wid * BR, BR)
            pltpu.async_copy(edges_hbm.at[kind, pl.ds(row0, BR)],
                             ebufs[0], dsem)

        @pl.loop(0, NBLK_MAX, step=2)
        def _(i):
            for p in range(2):
                iv = i + p

                @pl.when(iv < n)
                def _():
                    ebuf = ebufs[p]
                    pltpu.make_async_copy(
                        edges_hbm.at[kind, pl.ds(0, BR)], ebuf, dsem).wait()

                    @pl.when(iv + 1 < n)
                    def _():
                        row0 = pl.multiple_of((wid + (iv + 1) * NW) * BR, BR)
                        pltpu.async_copy(
                            edges_hbm.at[kind, pl.ds(row0, BR)],
                            ebufs[1 - p], dsem)

                    for j in range(BR):
                        pltpu.async_copy(ones_v, sh.at[ebuf.at[j]], ssem,
                                         add=True)
                    for j in range(BR):
                        pltpu.make_async_copy(ones_v, sh.at[ebuf.at[0]],
                                              ssem).wait()

        @pl.when(wid < REM_ROWS)
        def _():
            pltpu.sync_copy(edges_hbm.at[kind, NBLK * BR + wid], rbuf)
            pltpu.sync_copy(ones_v, sh.at[rbuf], add=True)

    plsc.subcore_barrier()

    sl = pl.ds(sid * HSLICE, HSLICE)
    pltpu.sync_copy(sh_src.at[sl], hist_hbm.at[2 * cid, sl])
    pltpu.sync_copy(sh_tgt.at[sl], hist_hbm.at[2 * cid + 1, sl])


_hist_call = pl.kernel(
    _hist_body,
    out_type=jax.ShapeDtypeStruct((2 * NC, HIST_PAD), jnp.int32),
    mesh=_mesh(),
    scratch_types=[
        pltpu.VMEM((HSLICE,), jnp.int32),
        pltpu.VMEM((BR, 128), jnp.int32),
        pltpu.VMEM((BR, 128), jnp.int32),
        pltpu.VMEM((128,), jnp.int32),
        pltpu.VMEM((128,), jnp.int32),
        pltpu.VMEM_SHARED((HIST_PAD,), jnp.int32),
        pltpu.VMEM_SHARED((HIST_PAD,), jnp.int32),
        pltpu.SemaphoreType.DMA,
        pltpu.SemaphoreType.DMA,
    ],
)


def _lookup_body(hist_hbm, in_emb, out_emb, off_hbm, out_hbm,
                 t4a, t4b, ia0, ia1, ib0, ib1, ra0, ra1, rb0, rb1, offv,
                 h0, h1, g0, g1, o0, o1):
    cid = lax.axis_index("c")
    sid = lax.axis_index("s")
    wid = sid * NC + cid

    t4s = (t4a, t4b)
    idx_in = (ia0, ia1)
    idx_out = (ib0, ib1)
    rows_a = (ra0, ra1)
    rows_b = (rb0, rb1)
    hsem = (h0, h1)
    gsem = (g0, g1)
    osem = (o0, o1)

    pltpu.sync_copy(off_hbm, offv)
    off_vec = offv[...]

    nfull = NCHUNK // NW
    n = jnp.where(wid < NCHUNK - nfull * NW, nfull + 1, nfull)

    @pl.when(n > 0)
    def _():
        base = pl.multiple_of(wid * 128, 128)
        pltpu.async_copy(hist_hbm.at[:, pl.ds(base, 128)], t4s[0], h0)

    @pl.loop(0, NCHUNK_MAX + 1, step=2)
    def _(i):
        for p in range(2):
            iv = i + p
            q = 1 - p

            @pl.when(iv < n)
            def _():
                pltpu.make_async_copy(
                    hist_hbm.at[:, pl.ds(0, 128)], t4s[p], hsem[p]).wait()

                @pl.when(iv + 1 < n)
                def _():
                    base = pl.multiple_of((wid + (iv + 1) * NW) * 128, 128)
                    pltpu.async_copy(hist_hbm.at[:, pl.ds(base, 128)],
                                     t4s[q], hsem[q])

                t4 = t4s[p]
                for j in range(128 // L):
                    sl = pl.ds(j * L, L)
                    v = t4[1, sl] + t4[3, sl] + off_vec
                    idx_in[p][sl] = jnp.minimum(jnp.maximum(v, 0), MAX_DEGREE)
                    w = t4[0, sl] + t4[2, sl] + off_vec
                    idx_out[p][sl] = jnp.minimum(jnp.maximum(w, 0), MAX_DEGREE)

                @pl.when(iv >= 2)
                def _():
                    pltpu.make_async_copy(
                        rows_a[p], out_hbm.at[pl.ds(0, 128)], osem[p]).wait()

                pltpu.async_copy(in_emb.at[idx_in[p]], rows_a[p], gsem[p])
                pltpu.async_copy(out_emb.at[idx_out[p]], rows_b[p], gsem[p])

            ivm = iv - 1

            @pl.when((ivm >= 0) & (ivm < n))
            def _():
                gm = wid + ivm * NW
                pltpu.make_async_copy(
                    in_emb.at[idx_in[q]], rows_a[q], gsem[q]).wait()
                pltpu.make_async_copy(
                    out_emb.at[idx_out[q]], rows_b[q], gsem[q]).wait()

                @pl.loop(0, 128)
                def _(r):
                    for j in range(HIDDEN_DIM // L):
                        sl = pl.ds(j * L, L)
                        rows_a[q][r, sl] = rows_a[q][r, sl] + rows_b[q][r, sl]

                @pl.when(gm < TAIL_CHUNK)
                def _():
                    obase = pl.multiple_of(gm * 128, 128)
                    pltpu.async_copy(rows_a[q],
                                     out_hbm.at[pl.ds(obase, 128)], osem[q])

                @pl.when(gm == TAIL_CHUNK)
                def _():
                    pltpu.async_copy(rows_a[q].at[pl.ds(0, TAIL_N)],
                                     out_hbm.at[pl.ds(TAIL_BASE, TAIL_N)],
                                     osem[q])

    @pl.when(wid == TAIL_WID)
    def _():
        pltpu.make_async_copy(rows_a[0].at[pl.ds(0, TAIL_N)],
                              out_hbm.at[pl.ds(TAIL_BASE, TAIL_N)], o0).wait()

    @pl.when(wid != TAIL_WID)
    def _():
        pltpu.make_async_copy(rows_a[0], out_hbm.at[pl.ds(0, 128)], o0).wait()

    pltpu.make_async_copy(rows_a[1], out_hbm.at[pl.ds(0, 128)], o1).wait()


_lookup_call = pl.kernel(
    _lookup_body,
    out_type=jax.ShapeDtypeStruct((N_NODES, HIDDEN_DIM), jnp.float32),
    mesh=_mesh(),
    scratch_types=[
        pltpu.VMEM((2 * NC, 128), jnp.int32),
        pltpu.VMEM((2 * NC, 128), jnp.int32),
        pltpu.VMEM((128,), jnp.int32),
        pltpu.VMEM((128,), jnp.int32),
        pltpu.VMEM((128,), jnp.int32),
        pltpu.VMEM((128,), jnp.int32),
        pltpu.VMEM((128, HIDDEN_DIM), jnp.float32),
        pltpu.VMEM((128, HIDDEN_DIM), jnp.float32),
        pltpu.VMEM((128, HIDDEN_DIM), jnp.float32),
        pltpu.VMEM((128, HIDDEN_DIM), jnp.float32),
        pltpu.VMEM((L,), jnp.int32),
        pltpu.SemaphoreType.DMA,
        pltpu.SemaphoreType.DMA,
        pltpu.SemaphoreType.DMA,
        pltpu.SemaphoreType.DMA,
        pltpu.SemaphoreType.DMA,
        pltpu.SemaphoreType.DMA,
    ],
)


def kernel(edge_index, n_nodes, in_embed, out_embed):
    edges = edge_index.reshape(2, EROWS, 128)
    off = (jnp.asarray(n_nodes) - N_NODES).astype(jnp.int32)
    off_v = jnp.full((L,), off, jnp.int32)
    hist = _hist_call(edges)
    return _lookup_call(hist, in_embed, out_embed, off_v)


# speedup vs baseline: 1.5866x; 1.0038x over previous
import functools

import jax
import jax.numpy as jnp
from jax import lax
from jax.experimental import pallas as pl
from jax.experimental.pallas import tpu as pltpu
from jax.experimental.pallas import tpu_sc as plsc

MAX_DEGREE = 512
HIDDEN_DIM = 128
N_NODES = 100000
N_EDGES = 1600000

NC = 2
NS = 16
NW = NC * NS
L = 16

EROWS = N_EDGES // 128
BR = 32
NBLK = EROWS // BR
REM_ROWS = EROWS - NBLK * BR
NBLK_MAX = -(-NBLK // NW) + 1
HIST_PAD = 100352
HSLICE = HIST_PAD // NS
C = 64
D = 6
NCHUNK = -(-N_NODES // C)
NCHUNK_MAX = -(-NCHUNK // NW)
TAIL_CHUNK = NCHUNK - 1
TAIL_WID = TAIL_CHUNK % NW
TAIL_BASE = TAIL_CHUNK * C
TAIL_N = N_NODES - TAIL_BASE
NV = NCHUNK_MAX + D - 1

_mesh = functools.partial(
    plsc.VectorSubcoreMesh,
    core_axis_name="c", subcore_axis_name="s",
    num_cores=NC, num_subcores=NS,
)


def _hist_body(edges_hbm, hist_hbm, zbuf, eb0, eb1, ones_v, rbuf,
               sh_src, sh_tgt, dsem, ssem):
    cid = lax.axis_index("c")
    sid = lax.axis_index("s")
    wid = sid * NC + cid

    @pl.loop(0, HSLICE // L)
    def _(i):
        zbuf[pl.ds(i * L, L)] = jnp.zeros((L,), jnp.int32)

    for j in range(128 // L):
        ones_v[pl.ds(j * L, L)] = jnp.ones((L,), jnp.int32)

    pltpu.sync_copy(zbuf, sh_src.at[pl.ds(sid * HSLICE, HSLICE)])
    pltpu.sync_copy(zbuf, sh_tgt.at[pl.ds(sid * HSLICE, HSLICE)])
    plsc.subcore_barrier()

    nfull = NBLK // NW
    n = jnp.where(wid < NBLK - nfull * NW, nfull + 1, nfull)
    ebufs = (eb0, eb1)

    for kind, sh in ((0, sh_src), (1, sh_tgt)):
        @pl.when(n > 0)
        def _():
            row0 = pl.multiple_of(wid * BR, BR)
            pltpu.async_copy(edges_hbm.at[kind, pl.ds(row0, BR)],
                             ebufs[0], dsem)

        @pl.loop(0, NBLK_MAX, step=2)
        def _(i):
            for p in range(2):
                iv = i + p

                @pl.when(iv < n)
                def _():
                    ebuf = ebufs[p]
                    pltpu.make_async_copy(
                        edges_hbm.at[kind, pl.ds(0, BR)], ebuf, dsem).wait()

                    @pl.when(iv + 1 < n)
                    def _():
                        row0 = pl.multiple_of((wid + (iv + 1) * NW) * BR, BR)
                        pltpu.async_copy(
                            edges_hbm.at[kind, pl.ds(row0, BR)],
                            ebufs[1 - p], dsem)

                    for j in range(BR):
                        pltpu.async_copy(ones_v, sh.at[ebuf.at[j]], ssem,
                                         add=True)
                    for j in range(BR):
                        pltpu.make_async_copy(ones_v, sh.at[ebuf.at[0]],
                                              ssem).wait()

        @pl.when(wid < REM_ROWS)
        def _():
            pltpu.sync_copy(edges_hbm.at[kind, NBLK * BR + wid], rbuf)
            pltpu.sync_copy(ones_v, sh.at[rbuf], add=True)

    plsc.subcore_barrier()

    sl = pl.ds(sid * HSLICE, HSLICE)
    pltpu.sync_copy(sh_src.at[sl], hist_hbm.at[2 * cid, sl])
    pltpu.sync_copy(sh_tgt.at[sl], hist_hbm.at[2 * cid + 1, sl])


_hist_call = pl.kernel(
    _hist_body,
    out_type=jax.ShapeDtypeStruct((2 * NC, HIST_PAD), jnp.int32),
    mesh=_mesh(),
    scratch_types=[
        pltpu.VMEM((HSLICE,), jnp.int32),
        pltpu.VMEM((BR, 128), jnp.int32),
        pltpu.VMEM((BR, 128), jnp.int32),
        pltpu.VMEM((128,), jnp.int32),
        pltpu.VMEM((128,), jnp.int32),
        pltpu.VMEM_SHARED((HIST_PAD,), jnp.int32),
        pltpu.VMEM_SHARED((HIST_PAD,), jnp.int32),
        pltpu.SemaphoreType.DMA,
        pltpu.SemaphoreType.DMA,
    ],
)


def _lookup_body(hist_hbm, in_emb, out_emb, off_hbm, out_hbm,
                 t4s, idx_in, idx_out, rows_a, rows_b, offv,
                 hsem, gsem, osem):
    cid = lax.axis_index("c")
    sid = lax.axis_index("s")
    wid = sid * NC + cid

    pltpu.sync_copy(off_hbm, offv)
    off_vec = offv[...]

    nfull = NCHUNK // NW
    n = jnp.where(wid < NCHUNK - nfull * NW, nfull + 1, nfull)

    def _fetch_hist(g, p):
        base = pl.multiple_of(g * C, C)
        for j in range(4):
            pltpu.async_copy(hist_hbm.at[j, pl.ds(base, C)],
                             t4s[p].at[pl.ds(j * C, C)], hsem[p])

    def _wait_hist(p):
        for j in range(4):
            pltpu.make_async_copy(hist_hbm.at[0, pl.ds(0, C)],
                                  t4s[p].at[pl.ds(0, C)], hsem[p]).wait()

    @pl.when(n > 0)
    def _():
        _fetch_hist(wid, 0)

    @pl.loop(0, NV, step=D)
    def _(i):
        for p in range(D):
            iv = i + p

            @pl.when(iv < n)
            def _():
                _wait_hist(p)

                pn = (p + 1) % D

                @pl.when(iv + 1 < n)
                def _():
                    _fetch_hist(wid + (iv + 1) * NW, pn)

                t4 = t4s[p]
                for j in range(C // L):
                    sl = pl.ds(j * L, L)
                    v = t4[pl.ds(1 * C + j * L, L)] + \
                        t4[pl.ds(3 * C + j * L, L)] + off_vec
                    idx_in[p][sl] = jnp.minimum(jnp.maximum(v, 0), MAX_DEGREE)
                    w = t4[pl.ds(0 * C + j * L, L)] + \
                        t4[pl.ds(2 * C + j * L, L)] + off_vec
                    idx_out[p][sl] = jnp.minimum(jnp.maximum(w, 0), MAX_DEGREE)

                @pl.when(iv >= D)
                def _():
                    pltpu.make_async_copy(
                        rows_a[p], out_hbm.at[pl.ds(0, C)], osem[p]).wait()

                pltpu.async_copy(in_emb.at[idx_in[p]], rows_a[p], gsem[p])
                pltpu.async_copy(out_emb.at[idx_out[p]], rows_b[p], gsem[p])

            ivm = iv - (D - 1)
            q = (p + 1) % D

            @pl.when((ivm >= 0) & (ivm < n))
            def _():
                gm = wid + ivm * NW
                pltpu.make_async_copy(
                    in_emb.at[idx_in[q]], rows_a[q], gsem[q]).wait()
                pltpu.make_async_copy(
                    out_emb.at[idx_out[q]], rows_b[q], gsem[q]).wait()

                @pl.loop(0, C)
                def _(r):
                    for j in range(HIDDEN_DIM // L):
                        sl = pl.ds(j * L, L)
                        rows_a[q][r, sl] = rows_a[q][r, sl] + rows_b[q][r, sl]

                @pl.when(gm < TAIL_CHUNK)
                def _():
                    obase = pl.multiple_of(gm * C, C)
                    pltpu.async_copy(rows_a[q],
                                     out_hbm.at[pl.ds(obase, C)], osem[q])

                @pl.when(gm == TAIL_CHUNK)
                def _():
                    pltpu.async_copy(rows_a[q].at[pl.ds(0, TAIL_N)],
                                     out_hbm.at[pl.ds(TAIL_BASE, TAIL_N)],
                                     osem[q])

    TAIL_PARITY = (NCHUNK_MAX - 1) % D
    for p in range(D):
        if p == TAIL_PARITY:
            @pl.when(wid == TAIL_WID)
            def _():
                pltpu.make_async_copy(
                    rows_a[p].at[pl.ds(0, TAIL_N)],
                    out_hbm.at[pl.ds(TAIL_BASE, TAIL_N)], osem[p]).wait()

            @pl.when(wid != TAIL_WID)
            def _():
                pltpu.make_async_copy(
                    rows_a[p], out_hbm.at[pl.ds(0, C)], osem[p]).wait()
        else:
            pltpu.make_async_copy(
                rows_a[p], out_hbm.at[pl.ds(0, C)], osem[p]).wait()


_lookup_call = pl.kernel(
    _lookup_body,
    out_type=jax.ShapeDtypeStruct((N_NODES, HIDDEN_DIM), jnp.float32),
    mesh=_mesh(),
    scratch_types=[
        [pltpu.VMEM((2 * NC * C,), jnp.int32) for _ in range(D)],
        [pltpu.VMEM((C,), jnp.int32) for _ in range(D)],
        [pltpu.VMEM((C,), jnp.int32) for _ in range(D)],
        [pltpu.VMEM((C, HIDDEN_DIM), jnp.float32) for _ in range(D)],
        [pltpu.VMEM((C, HIDDEN_DIM), jnp.float32) for _ in range(D)],
        pltpu.VMEM((L,), jnp.int32),
        [pltpu.SemaphoreType.DMA for _ in range(D)],
        [pltpu.SemaphoreType.DMA for _ in range(D)],
        [pltpu.SemaphoreType.DMA for _ in range(D)],
    ],
)


def kernel(edge_index, n_nodes, in_embed, out_embed):
    edges = edge_index.reshape(2, EROWS, 128)
    off = (jnp.asarray(n_nodes) - N_NODES).astype(jnp.int32)
    off_v = jnp.full((L,), off, jnp.int32)
    hist = _hist_call(edges)
    return _lookup_call(hist, in_embed, out_embed, off_v)


# speedup vs baseline: 7.9197x; 4.9918x over previous
import functools

import jax
import jax.numpy as jnp
from jax import lax
from jax.experimental import pallas as pl
from jax.experimental.pallas import tpu as pltpu
from jax.experimental.pallas import tpu_sc as plsc

MAX_DEGREE = 512
HIDDEN_DIM = 128
N_NODES = 100000
N_EDGES = 1600000

NC = 2
NS = 16
NW = NC * NS
L = 16

EROWS = N_EDGES // 128
BR = 32
NBLK = EROWS // BR
REM_ROWS = EROWS - NBLK * BR
NBLK_MAX = -(-NBLK // NW) + 1
HIST_PAD = 100352
HSLICE = HIST_PAD // NS
C = 64
D = 6
NCHUNK = -(-N_NODES // C)
NCHUNK_MAX = -(-NCHUNK // NW)
TAIL_CHUNK = NCHUNK - 1
TAIL_WID = TAIL_CHUNK % NW
TAIL_BASE = TAIL_CHUNK * C
TAIL_N = N_NODES - TAIL_BASE
NV = NCHUNK_MAX + D - 1

_mesh = functools.partial(
    plsc.VectorSubcoreMesh,
    core_axis_name="c", subcore_axis_name="s",
    num_cores=NC, num_subcores=NS,
)


def _hist_body(edges_hbm, hist_hbm, zbuf, eb0, eb1, ones_v, rbuf,
               sh_src, sh_tgt, dsem, ssem):
    cid = lax.axis_index("c")
    sid = lax.axis_index("s")
    wid = sid * NC + cid

    @pl.loop(0, HSLICE // L)
    def _(i):
        zbuf[pl.ds(i * L, L)] = jnp.zeros((L,), jnp.int32)

    for j in range(128 // L):
        ones_v[pl.ds(j * L, L)] = jnp.ones((L,), jnp.int32)

    pltpu.sync_copy(zbuf, sh_src.at[pl.ds(sid * HSLICE, HSLICE)])
    pltpu.sync_copy(zbuf, sh_tgt.at[pl.ds(sid * HSLICE, HSLICE)])
    plsc.subcore_barrier()

    nfull = NBLK // NW
    n = jnp.where(wid < NBLK - nfull * NW, nfull + 1, nfull)
    ebufs = (eb0, eb1)

    for kind, sh in ((0, sh_src), (1, sh_tgt)):
        @pl.when(n > 0)
        def _():
            row0 = pl.multiple_of(wid * BR, BR)
            pltpu.async_copy(edges_hbm.at[kind, pl.ds(row0, BR)],
                             ebufs[0], dsem)

        @pl.loop(0, NBLK_MAX, step=2)
        def _(i):
            for p in range(2):
                iv = i + p

                @pl.when(iv < n)
                def _():
                    ebuf = ebufs[p]
                    pltpu.make_async_copy(
                        edges_hbm.at[kind, pl.ds(0, BR)], ebuf, dsem).wait()

                    @pl.when(iv + 1 < n)
                    def _():
                        row0 = pl.multiple_of((wid + (iv + 1) * NW) * BR, BR)
                        pltpu.async_copy(
                            edges_hbm.at[kind, pl.ds(row0, BR)],
                            ebufs[1 - p], dsem)

                    for j in range(BR):
                        pltpu.async_copy(ones_v, sh.at[ebuf.at[j]], ssem,
                                         add=True)
                    for j in range(BR):
                        pltpu.make_async_copy(ones_v, sh.at[ebuf.at[0]],
                                              ssem).wait()

        @pl.when(wid < REM_ROWS)
        def _():
            pltpu.sync_copy(edges_hbm.at[kind, NBLK * BR + wid], rbuf)
            pltpu.sync_copy(ones_v, sh.at[rbuf], add=True)

    plsc.subcore_barrier()

    sl = pl.ds(sid * HSLICE, HSLICE)
    pltpu.sync_copy(sh_src.at[sl], hist_hbm.at[2 * cid, sl])
    pltpu.sync_copy(sh_tgt.at[sl], hist_hbm.at[2 * cid + 1, sl])


_hist_call = pl.kernel(
    _hist_body,
    out_type=jax.ShapeDtypeStruct((2 * NC, HIST_PAD), jnp.int32),
    mesh=_mesh(),
    scratch_types=[
        pltpu.VMEM((HSLICE,), jnp.int32),
        pltpu.VMEM((BR, 128), jnp.int32),
        pltpu.VMEM((BR, 128), jnp.int32),
        pltpu.VMEM((128,), jnp.int32),
        pltpu.VMEM((128,), jnp.int32),
        pltpu.VMEM_SHARED((HIST_PAD,), jnp.int32),
        pltpu.VMEM_SHARED((HIST_PAD,), jnp.int32),
        pltpu.SemaphoreType.DMA,
        pltpu.SemaphoreType.DMA,
    ],
)


TAB_ROWS = 2 * (MAX_DEGREE + 1)
TROWS_PER_TILE = TAB_ROWS // NS
TROWS_REM = TAB_ROWS - TROWS_PER_TILE * NS


def _lookup_body(tab_hbm, hist_hbm, off_hbm, out_hbm,
                 t4s, idx_in, idx_out, rows_a, rows_b, offv, sh_tab,
                 hsem, gsem, osem):
    cid = lax.axis_index("c")
    sid = lax.axis_index("s")
    wid = sid * NC + cid

    pltpu.sync_copy(off_hbm, offv)
    off_vec = offv[...]

    trow = pl.multiple_of(sid * TROWS_PER_TILE, 8)
    pltpu.sync_copy(tab_hbm.at[pl.ds(trow, TROWS_PER_TILE)],
                    sh_tab.at[pl.ds(trow, TROWS_PER_TILE)])

    @pl.when(sid == 0)
    def _():
        base = TROWS_PER_TILE * NS
        pltpu.sync_copy(tab_hbm.at[pl.ds(base, TROWS_REM)],
                        sh_tab.at[pl.ds(base, TROWS_REM)])

    plsc.subcore_barrier()

    nfull = NCHUNK // NW
    n = jnp.where(wid < NCHUNK - nfull * NW, nfull + 1, nfull)

    def _fetch_hist(g, p):
        base = pl.multiple_of(g * C, C)
        for j in range(4):
            pltpu.async_copy(hist_hbm.at[j, pl.ds(base, C)],
                             t4s[p].at[pl.ds(j * C, C)], hsem[p])

    def _wait_hist(p):
        for j in range(4):
            pltpu.make_async_copy(hist_hbm.at[0, pl.ds(0, C)],
                                  t4s[p].at[pl.ds(0, C)], hsem[p]).wait()

    @pl.when(n > 0)
    def _():
        _fetch_hist(wid, 0)

    @pl.loop(0, NV, step=D)
    def _(i):
        for p in range(D):
            iv = i + p

            @pl.when(iv < n)
            def _():
                _wait_hist(p)

                pn = (p + 1) % D

                @pl.when(iv + 1 < n)
                def _():
                    _fetch_hist(wid + (iv + 1) * NW, pn)

                t4 = t4s[p]
                for j in range(C // L):
                    sl = pl.ds(j * L, L)
                    v = t4[pl.ds(1 * C + j * L, L)] + \
                        t4[pl.ds(3 * C + j * L, L)] + off_vec
                    idx_in[p][sl] = jnp.minimum(jnp.maximum(v, 0), MAX_DEGREE)
                    w = t4[pl.ds(0 * C + j * L, L)] + \
                        t4[pl.ds(2 * C + j * L, L)] + off_vec
                    idx_out[p][sl] = (jnp.minimum(jnp.maximum(w, 0),
                                                  MAX_DEGREE)
                                      + (MAX_DEGREE + 1))

                @pl.when(iv >= D)
                def _():
                    pltpu.make_async_copy(
                        rows_a[p], out_hbm.at[pl.ds(0, C)], osem[p]).wait()

                pltpu.async_copy(sh_tab.at[idx_in[p]], rows_a[p], gsem[p])
                pltpu.async_copy(sh_tab.at[idx_out[p]], rows_b[p], gsem[p])

            ivm = iv - (D - 1)
            q = (p + 1) % D

            @pl.when((ivm >= 0) & (ivm < n))
            def _():
                gm = wid + ivm * NW
                pltpu.make_async_copy(
                    sh_tab.at[idx_in[q]], rows_a[q], gsem[q]).wait()
                pltpu.make_async_copy(
                    sh_tab.at[idx_out[q]], rows_b[q], gsem[q]).wait()

                @pl.loop(0, C)
                def _(r):
                    for j in range(HIDDEN_DIM // L):
                        sl = pl.ds(j * L, L)
                        rows_a[q][r, sl] = rows_a[q][r, sl] + rows_b[q][r, sl]

                @pl.when(gm < TAIL_CHUNK)
                def _():
                    obase = pl.multiple_of(gm * C, C)
                    pltpu.async_copy(rows_a[q],
                                     out_hbm.at[pl.ds(obase, C)], osem[q])

                @pl.when(gm == TAIL_CHUNK)
                def _():
                    pltpu.async_copy(rows_a[q].at[pl.ds(0, TAIL_N)],
                                     out_hbm.at[pl.ds(TAIL_BASE, TAIL_N)],
                                     osem[q])

    TAIL_PARITY = (NCHUNK_MAX - 1) % D
    for p in range(D):
        if p == TAIL_PARITY:
            @pl.when(wid == TAIL_WID)
            def _():
                pltpu.make_async_copy(
                    rows_a[p].at[pl.ds(0, TAIL_N)],
                    out_hbm.at[pl.ds(TAIL_BASE, TAIL_N)], osem[p]).wait()

            @pl.when(wid != TAIL_WID)
            def _():
                pltpu.make_async_copy(
                    rows_a[p], out_hbm.at[pl.ds(0, C)], osem[p]).wait()
        else:
            pltpu.make_async_copy(
                rows_a[p], out_hbm.at[pl.ds(0, C)], osem[p]).wait()


_lookup_call = pl.kernel(
    _lookup_body,
    out_type=jax.ShapeDtypeStruct((N_NODES, HIDDEN_DIM), jnp.float32),
    mesh=_mesh(),
    scratch_types=[
        [pltpu.VMEM((2 * NC * C,), jnp.int32) for _ in range(D)],
        [pltpu.VMEM((C,), jnp.int32) for _ in range(D)],
        [pltpu.VMEM((C,), jnp.int32) for _ in range(D)],
        [pltpu.VMEM((C, HIDDEN_DIM), jnp.float32) for _ in range(D)],
        [pltpu.VMEM((C, HIDDEN_DIM), jnp.float32) for _ in range(D)],
        pltpu.VMEM((L,), jnp.int32),
        pltpu.VMEM_SHARED((TAB_ROWS, HIDDEN_DIM), jnp.float32),
        [pltpu.SemaphoreType.DMA for _ in range(D)],
        [pltpu.SemaphoreType.DMA for _ in range(D)],
        [pltpu.SemaphoreType.DMA for _ in range(D)],
    ],
)


def kernel(edge_index, n_nodes, in_embed, out_embed):
    edges = edge_index.reshape(2, EROWS, 128)
    off = (jnp.asarray(n_nodes) - N_NODES).astype(jnp.int32)
    off_v = jnp.full((L,), off, jnp.int32)
    tab = jnp.concatenate([in_embed, out_embed], axis=0)
    hist = _hist_call(edges)
    return _lookup_call(tab, hist, off_v)


# speedup vs baseline: 9.6004x; 1.2122x over previous
import functools

import jax
import jax.numpy as jnp
from jax import lax
from jax.experimental import pallas as pl
from jax.experimental.pallas import tpu as pltpu
from jax.experimental.pallas import tpu_sc as plsc

MAX_DEGREE = 512
HIDDEN_DIM = 128
N_NODES = 100000
N_EDGES = 1600000

NC = 2
NS = 16
NW = NC * NS
L = 16

EROWS = N_EDGES // 128
BR = 32
BW = BR * 128
NBLK = EROWS // BR
REM_ROWS = EROWS - NBLK * BR
NBLK_MAX = -(-NBLK // NW) + 1
HIST_PAD = 100352
HSLICE = HIST_PAD // NS
C = 64
D = 6
NCHUNK = -(-N_NODES // C)
NCHUNK_MAX = -(-NCHUNK // NW)
TAIL_CHUNK = NCHUNK - 1
TAIL_WID = TAIL_CHUNK % NW
TAIL_BASE = TAIL_CHUNK * C
TAIL_N = N_NODES - TAIL_BASE
NV = NCHUNK_MAX + D - 1

TAB_ROWS = MAX_DEGREE + 1
OUT_TAB = 520
SH_TAB_ROWS = OUT_TAB + TAB_ROWS

_mesh = functools.partial(
    plsc.VectorSubcoreMesh,
    core_axis_name="c", subcore_axis_name="s",
    num_cores=NC, num_subcores=NS,
)


def _hist_body(edges_hbm, hist_hbm, zbuf, eb0, eb1, ones_v, rbuf,
               sh_src, sh_tgt, dsem, ssem):
    cid = lax.axis_index("c")
    sid = lax.axis_index("s")
    wid = sid * NC + cid

    @pl.loop(0, HSLICE // L)
    def _(i):
        zbuf[pl.ds(i * L, L)] = jnp.zeros((L,), jnp.int32)

    for j in range(128 // L):
        ones_v[pl.ds(j * L, L)] = jnp.ones((L,), jnp.int32)

    pltpu.sync_copy(zbuf, sh_src.at[pl.ds(sid * HSLICE, HSLICE)])
    pltpu.sync_copy(zbuf, sh_tgt.at[pl.ds(sid * HSLICE, HSLICE)])
    plsc.subcore_barrier()

    nfull = NBLK // NW
    n = jnp.where(wid < NBLK - nfull * NW, nfull + 1, nfull)
    ebufs = (eb0, eb1)

    for kind, sh in ((0, sh_src), (1, sh_tgt)):
        @pl.when(n > 0)
        def _():
            off0 = pl.multiple_of(wid * BW, BW)
            pltpu.async_copy(edges_hbm.at[kind, pl.ds(off0, BW)],
                             ebufs[0], dsem)

        @pl.loop(0, NBLK_MAX, step=2)
        def _(i):
            for p in range(2):
                iv = i + p

                @pl.when(iv < n)
                def _():
                    ebuf = ebufs[p]
                    pltpu.make_async_copy(
                        edges_hbm.at[kind, pl.ds(0, BW)], ebuf, dsem).wait()

                    @pl.when(iv + 1 < n)
                    def _():
                        off = pl.multiple_of((wid + (iv + 1) * NW) * BW, BW)
                        pltpu.async_copy(
                            edges_hbm.at[kind, pl.ds(off, BW)],
                            ebufs[1 - p], dsem)

                    for j in range(BR):
                        pltpu.async_copy(
                            ones_v, sh.at[ebuf.at[pl.ds(j * 128, 128)]],
                            ssem, add=True)
                    for j in range(BR):
                        pltpu.make_async_copy(
                            ones_v, sh.at[ebuf.at[pl.ds(0, 128)]],
                            ssem).wait()

        @pl.when(wid < REM_ROWS)
        def _():
            roff = pl.multiple_of((NBLK * BR + wid) * 128, 128)
            pltpu.sync_copy(edges_hbm.at[kind, pl.ds(roff, 128)], rbuf)
            pltpu.sync_copy(ones_v, sh.at[rbuf], add=True)

    plsc.subcore_barrier()

    for kind, sh in ((0, sh_src), (1, sh_tgt)):
        src_sl = pl.ds(sid * HSLICE, HSLICE)
        doff = pl.multiple_of((2 * cid + kind) * HIST_PAD + sid * HSLICE,
                              HSLICE)
        pltpu.sync_copy(sh.at[src_sl], hist_hbm.at[pl.ds(doff, HSLICE)])


_hist_call = pl.kernel(
    _hist_body,
    out_type=jax.ShapeDtypeStruct((2 * NC * HIST_PAD,), jnp.int32),
    mesh=_mesh(),
    scratch_types=[
        pltpu.VMEM((HSLICE,), jnp.int32),
        pltpu.VMEM((BW,), jnp.int32),
        pltpu.VMEM((BW,), jnp.int32),
        pltpu.VMEM((128,), jnp.int32),
        pltpu.VMEM((128,), jnp.int32),
        pltpu.VMEM_SHARED((HIST_PAD,), jnp.int32),
        pltpu.VMEM_SHARED((HIST_PAD,), jnp.int32),
        pltpu.SemaphoreType.DMA,
        pltpu.SemaphoreType.DMA,
    ],
)


def _lookup_body(in_emb, out_emb, hist_hbm, off_hbm, out_hbm,
                 t4s, idx_in, idx_out, rows_a, rows_b, offv, sh_tab,
                 hsem, gsem, osem):
    cid = lax.axis_index("c")
    sid = lax.axis_index("s")
    wid = sid * NC + cid

    pltpu.sync_copy(off_hbm, offv)
    off_vec = offv[...]

    for base, tab in ((0, in_emb), (OUT_TAB, out_emb)):
        @pl.when(sid < 12)
        def _():
            r0 = pl.multiple_of(sid * 40, 8)
            pltpu.sync_copy(tab.at[pl.ds(r0, 40)],
                            sh_tab.at[pl.ds(base + r0, 40)])

        @pl.when(sid == 12)
        def _():
            pltpu.sync_copy(tab.at[pl.ds(480, TAB_ROWS - 480)],
                            sh_tab.at[pl.ds(base + 480, TAB_ROWS - 480)])

    plsc.subcore_barrier()

    nfull = NCHUNK // NW
    n = jnp.where(wid < NCHUNK - nfull * NW, nfull + 1, nfull)

    def _fetch_hist(g, p):
        for j in range(4):
            off = pl.multiple_of(j * HIST_PAD + g * C, 8)
            pltpu.async_copy(hist_hbm.at[pl.ds(off, C)],
                             t4s[p].at[pl.ds(j * C, C)], hsem[p])

    def _wait_hist(p):
        for j in range(4):
            pltpu.make_async_copy(hist_hbm.at[pl.ds(0, C)],
                                  t4s[p].at[pl.ds(0, C)], hsem[p]).wait()

    @pl.when(n > 0)
    def _():
        _fetch_hist(wid, 0)

    @pl.loop(0, NV, step=D)
    def _(i):
        for p in range(D):
            iv = i + p

            @pl.when(iv < n)
            def _():
                _wait_hist(p)

                pn = (p + 1) % D

                @pl.when(iv + 1 < n)
                def _():
                    _fetch_hist(wid + (iv + 1) * NW, pn)

                t4 = t4s[p]
                for j in range(C // L):
                    sl = pl.ds(j * L, L)
                    v = t4[pl.ds(1 * C + j * L, L)] + \
                        t4[pl.ds(3 * C + j * L, L)] + off_vec
                    idx_in[p][sl] = jnp.minimum(jnp.maximum(v, 0), MAX_DEGREE)
                    w = t4[pl.ds(0 * C + j * L, L)] + \
                        t4[pl.ds(2 * C + j * L, L)] + off_vec
                    idx_out[p][sl] = (jnp.minimum(jnp.maximum(w, 0),
                                                  MAX_DEGREE) + OUT_TAB)

                @pl.when(iv >= D)
                def _():
                    pltpu.make_async_copy(
                        rows_a[p], out_hbm.at[pl.ds(0, C)], osem[p]).wait()

                pltpu.async_copy(sh_tab.at[idx_in[p]], rows_a[p], gsem[p])
                pltpu.async_copy(sh_tab.at[idx_out[p]], rows_b[p], gsem[p])

            ivm = iv - (D - 1)
            q = (p + 1) % D

            @pl.when((ivm >= 0) & (ivm < n))
            def _():
                gm = wid + ivm * NW
                pltpu.make_async_copy(
                    sh_tab.at[idx_in[q]], rows_a[q], gsem[q]).wait()
                pltpu.make_async_copy(
                    sh_tab.at[idx_out[q]], rows_b[q], gsem[q]).wait()

                @pl.loop(0, C)
                def _(r):
                    for j in range(HIDDEN_DIM // L):
                        sl = pl.ds(j * L, L)
                        rows_a[q][r, sl] = rows_a[q][r, sl] + rows_b[q][r, sl]

                @pl.when(gm < TAIL_CHUNK)
                def _():
                    obase = pl.multiple_of(gm * C, C)
                    pltpu.async_copy(rows_a[q],
                                     out_hbm.at[pl.ds(obase, C)], osem[q])

                @pl.when(gm == TAIL_CHUNK)
                def _():
                    pltpu.async_copy(rows_a[q].at[pl.ds(0, TAIL_N)],
                                     out_hbm.at[pl.ds(TAIL_BASE, TAIL_N)],
                                     osem[q])

    TAIL_PARITY = (NCHUNK_MAX - 1) % D
    for p in range(D):
        if p == TAIL_PARITY:
            @pl.when(wid == TAIL_WID)
            def _():
                pltpu.make_async_copy(
                    rows_a[p].at[pl.ds(0, TAIL_N)],
                    out_hbm.at[pl.ds(TAIL_BASE, TAIL_N)], osem[p]).wait()

            @pl.when(wid != TAIL_WID)
            def _():
                pltpu.make_async_copy(
                    rows_a[p], out_hbm.at[pl.ds(0, C)], osem[p]).wait()
        else:
            pltpu.make_async_copy(
                rows_a[p], out_hbm.at[pl.ds(0, C)], osem[p]).wait()


_lookup_call = pl.kernel(
    _lookup_body,
    out_type=jax.ShapeDtypeStruct((N_NODES, HIDDEN_DIM), jnp.float32),
    mesh=_mesh(),
    scratch_types=[
        [pltpu.VMEM((2 * NC * C,), jnp.int32) for _ in range(D)],
        [pltpu.VMEM((C,), jnp.int32) for _ in range(D)],
        [pltpu.VMEM((C,), jnp.int32) for _ in range(D)],
        [pltpu.VMEM((C, HIDDEN_DIM), jnp.float32) for _ in range(D)],
        [pltpu.VMEM((C, HIDDEN_DIM), jnp.float32) for _ in range(D)],
        pltpu.VMEM((L,), jnp.int32),
        pltpu.VMEM_SHARED((SH_TAB_ROWS + 7, HIDDEN_DIM), jnp.float32),
        [pltpu.SemaphoreType.DMA for _ in range(D)],
        [pltpu.SemaphoreType.DMA for _ in range(D)],
        [pltpu.SemaphoreType.DMA for _ in range(D)],
    ],
)


def kernel(edge_index, n_nodes, in_embed, out_embed):
    off = (jnp.asarray(n_nodes) - N_NODES).astype(jnp.int32)
    off_v = jnp.full((L,), off, jnp.int32)
    hist = _hist_call(edge_index)
    return _lookup_call(in_embed, out_embed, hist, off_v)


# speedup vs baseline: 9.6278x; 1.0028x over previous
import functools

import jax
import jax.numpy as jnp
from jax import lax
from jax.experimental import pallas as pl
from jax.experimental.pallas import tpu as pltpu
from jax.experimental.pallas import tpu_sc as plsc

MAX_DEGREE = 512
HIDDEN_DIM = 128
N_NODES = 100000
N_EDGES = 1600000

NC = 2
NS = 16
NW = NC * NS
L = 16

EROWS = N_EDGES // 128
BR = 32
BW = BR * 128
NBLK = EROWS // BR
REM_ROWS = EROWS - NBLK * BR
NBLK_MAX = -(-NBLK // NW) + 1
HIST_PAD = 100352
HSLICE = HIST_PAD // NS
C = 64
D = 6
NCHUNK = -(-N_NODES // C)
NCHUNK_MAX = -(-NCHUNK // NW)
TAIL_CHUNK = NCHUNK - 1
TAIL_WID = TAIL_CHUNK % NW
TAIL_BASE = TAIL_CHUNK * C
TAIL_N = N_NODES - TAIL_BASE
NV = NCHUNK_MAX + D - 1

TAB_ROWS = MAX_DEGREE + 1
OUT_TAB = 520
SH_TAB_ROWS = OUT_TAB + TAB_ROWS

_mesh = functools.partial(
    plsc.VectorSubcoreMesh,
    core_axis_name="c", subcore_axis_name="s",
    num_cores=NC, num_subcores=NS,
)


def _hist_body(edges_hbm, hist_hbm, zbuf, eb0, eb1, ones_v, rbuf,
               sh_src, sh_tgt, dsem, ssem):
    cid = lax.axis_index("c")
    sid = lax.axis_index("s")
    wid = sid * NC + cid

    @pl.loop(0, HSLICE // L)
    def _(i):
        zbuf[pl.ds(i * L, L)] = jnp.zeros((L,), jnp.int32)

    for j in range(512 // L):
        ones_v[pl.ds(j * L, L)] = jnp.ones((L,), jnp.int32)

    pltpu.sync_copy(zbuf, sh_src.at[pl.ds(sid * HSLICE, HSLICE)])
    pltpu.sync_copy(zbuf, sh_tgt.at[pl.ds(sid * HSLICE, HSLICE)])
    plsc.subcore_barrier()

    nfull = NBLK // NW
    n = jnp.where(wid < NBLK - nfull * NW, nfull + 1, nfull)
    ebufs = (eb0, eb1)

    for kind, sh in ((0, sh_src), (1, sh_tgt)):
        @pl.when(n > 0)
        def _():
            off0 = pl.multiple_of(wid * BW, BW)
            pltpu.async_copy(edges_hbm.at[kind, pl.ds(off0, BW)],
                             ebufs[0], dsem)

        @pl.loop(0, NBLK_MAX, step=2)
        def _(i):
            for p in range(2):
                iv = i + p

                @pl.when(iv < n)
                def _():
                    ebuf = ebufs[p]
                    pltpu.make_async_copy(
                        edges_hbm.at[kind, pl.ds(0, BW)], ebuf, dsem).wait()

                    @pl.when(iv + 1 < n)
                    def _():
                        off = pl.multiple_of((wid + (iv + 1) * NW) * BW, BW)
                        pltpu.async_copy(
                            edges_hbm.at[kind, pl.ds(off, BW)],
                            ebufs[1 - p], dsem)

                    for j in range(BW // 512):
                        pltpu.async_copy(
                            ones_v, sh.at[ebuf.at[pl.ds(j * 512, 512)]],
                            ssem, add=True)
                    for j in range(BW // 512):
                        pltpu.make_async_copy(
                            ones_v, sh.at[ebuf.at[pl.ds(0, 512)]],
                            ssem).wait()

        @pl.when(wid < REM_ROWS)
        def _():
            roff = pl.multiple_of((NBLK * BR + wid) * 128, 128)
            pltpu.sync_copy(edges_hbm.at[kind, pl.ds(roff, 128)], rbuf)
            pltpu.sync_copy(ones_v.at[pl.ds(0, 128)], sh.at[rbuf], add=True)

    plsc.subcore_barrier()

    for kind, sh in ((0, sh_src), (1, sh_tgt)):
        src_sl = pl.ds(sid * HSLICE, HSLICE)
        doff = pl.multiple_of((2 * cid + kind) * HIST_PAD + sid * HSLICE,
                              HSLICE)
        pltpu.sync_copy(sh.at[src_sl], hist_hbm.at[pl.ds(doff, HSLICE)])


_hist_call = pl.kernel(
    _hist_body,
    out_type=jax.ShapeDtypeStruct((2 * NC * HIST_PAD,), jnp.int32),
    mesh=_mesh(),
    scratch_types=[
        pltpu.VMEM((HSLICE,), jnp.int32),
        pltpu.VMEM((BW,), jnp.int32),
        pltpu.VMEM((BW,), jnp.int32),
        pltpu.VMEM((512,), jnp.int32),
        pltpu.VMEM((128,), jnp.int32),
        pltpu.VMEM_SHARED((HIST_PAD,), jnp.int32),
        pltpu.VMEM_SHARED((HIST_PAD,), jnp.int32),
        pltpu.SemaphoreType.DMA,
        pltpu.SemaphoreType.DMA,
    ],
)


def _lookup_body(in_emb, out_emb, hist_hbm, off_hbm, out_hbm,
                 t4s, idx, rows, offv, sh_tab,
                 hsem, gsem, osem):
    cid = lax.axis_index("c")
    sid = lax.axis_index("s")
    wid = sid * NC + cid

    pltpu.sync_copy(off_hbm, offv)
    off_vec = offv[...]

    for base, tab in ((0, in_emb), (OUT_TAB, out_emb)):
        @pl.when(sid < 12)
        def _():
            r0 = pl.multiple_of(sid * 40, 8)
            pltpu.sync_copy(tab.at[pl.ds(r0, 40)],
                            sh_tab.at[pl.ds(base + r0, 40)])

        @pl.when(sid == 12)
        def _():
            pltpu.sync_copy(tab.at[pl.ds(480, TAB_ROWS - 480)],
                            sh_tab.at[pl.ds(base + 480, TAB_ROWS - 480)])

    plsc.subcore_barrier()

    nfull = NCHUNK // NW
    n = jnp.where(wid < NCHUNK - nfull * NW, nfull + 1, nfull)

    def _fetch_hist(g, p):
        for j in range(4):
            off = pl.multiple_of(j * HIST_PAD + g * C, 8)
            pltpu.async_copy(hist_hbm.at[pl.ds(off, C)],
                             t4s[p].at[pl.ds(j * C, C)], hsem[p])

    def _wait_hist(p):
        for j in range(4):
            pltpu.make_async_copy(hist_hbm.at[pl.ds(0, C)],
                                  t4s[p].at[pl.ds(0, C)], hsem[p]).wait()

    @pl.when(n > 0)
    def _():
        _fetch_hist(wid, 0)

    @pl.loop(0, NV, step=D)
    def _(i):
        for p in range(D):
            iv = i + p

            @pl.when(iv < n)
            def _():
                _wait_hist(p)

                pn = (p + 1) % D

                @pl.when(iv + 1 < n)
                def _():
                    _fetch_hist(wid + (iv + 1) * NW, pn)

                t4 = t4s[p]
                for j in range(C // L):
                    sl = pl.ds(j * L, L)
                    v = t4[pl.ds(1 * C + j * L, L)] + \
                        t4[pl.ds(3 * C + j * L, L)] + off_vec
                    idx[p][sl] = jnp.minimum(jnp.maximum(v, 0), MAX_DEGREE)
                    w = t4[pl.ds(0 * C + j * L, L)] + \
                        t4[pl.ds(2 * C + j * L, L)] + off_vec
                    idx[p][pl.ds(C + j * L, L)] = (
                        jnp.minimum(jnp.maximum(w, 0), MAX_DEGREE) + OUT_TAB)

                @pl.when(iv >= D)
                def _():
                    pltpu.make_async_copy(
                        rows[p].at[pl.ds(0, C)],
                        out_hbm.at[pl.ds(0, C)], osem[p]).wait()

                pltpu.async_copy(sh_tab.at[idx[p]], rows[p], gsem[p])

            ivm = iv - (D - 1)
            q = (p + 1) % D

            @pl.when((ivm >= 0) & (ivm < n))
            def _():
                gm = wid + ivm * NW
                pltpu.make_async_copy(
                    sh_tab.at[idx[q]], rows[q], gsem[q]).wait()

                @pl.loop(0, C)
                def _(r):
                    for j in range(HIDDEN_DIM // L):
                        sl = pl.ds(j * L, L)
                        rows[q][r, sl] = rows[q][r, sl] + rows[q][C + r, sl]

                @pl.when(gm < TAIL_CHUNK)
                def _():
                    obase = pl.multiple_of(gm * C, C)
                    pltpu.async_copy(rows[q].at[pl.ds(0, C)],
                                     out_hbm.at[pl.ds(obase, C)], osem[q])

                @pl.when(gm == TAIL_CHUNK)
                def _():
                    pltpu.async_copy(rows[q].at[pl.ds(0, TAIL_N)],
                                     out_hbm.at[pl.ds(TAIL_BASE, TAIL_N)],
                                     osem[q])

    TAIL_PARITY = (NCHUNK_MAX - 1) % D
    for p in range(D):
        if p == TAIL_PARITY:
            @pl.when(wid == TAIL_WID)
            def _():
                pltpu.make_async_copy(
                    rows[p].at[pl.ds(0, TAIL_N)],
                    out_hbm.at[pl.ds(TAIL_BASE, TAIL_N)], osem[p]).wait()

            @pl.when(wid != TAIL_WID)
            def _():
                pltpu.make_async_copy(
                    rows[p].at[pl.ds(0, C)],
                    out_hbm.at[pl.ds(0, C)], osem[p]).wait()
        else:
            pltpu.make_async_copy(
                rows[p].at[pl.ds(0, C)],
                out_hbm.at[pl.ds(0, C)], osem[p]).wait()


_lookup_call = pl.kernel(
    _lookup_body,
    out_type=jax.ShapeDtypeStruct((N_NODES, HIDDEN_DIM), jnp.float32),
    mesh=_mesh(),
    scratch_types=[
        [pltpu.VMEM((2 * NC * C,), jnp.int32) for _ in range(D)],
        [pltpu.VMEM((2 * C,), jnp.int32) for _ in range(D)],
        [pltpu.VMEM((2 * C, HIDDEN_DIM), jnp.float32) for _ in range(D)],
        pltpu.VMEM((L,), jnp.int32),
        pltpu.VMEM_SHARED((SH_TAB_ROWS + 7, HIDDEN_DIM), jnp.float32),
        [pltpu.SemaphoreType.DMA for _ in range(D)],
        [pltpu.SemaphoreType.DMA for _ in range(D)],
        [pltpu.SemaphoreType.DMA for _ in range(D)],
    ],
)


def kernel(edge_index, n_nodes, in_embed, out_embed):
    off = (jnp.asarray(n_nodes) - N_NODES).astype(jnp.int32)
    off_v = jnp.full((L,), off, jnp.int32)
    hist = _hist_call(edge_index)
    return _lookup_call(in_embed, out_embed, hist, off_v)
